# baseline (speedup 1.0000x reference)
import functools

import jax
import jax.numpy as jnp
from jax import lax
from jax.experimental import pallas as pl
from jax.experimental.pallas import tpu as pltpu
from jax.experimental.pallas import tpu_sc as plsc

BATCH = 16384
HIST = 50
EMBED_DIM = 64
VOCAB = 1000000

NUM_CORES = 2
NUM_SUBCORES = 16
NUM_WORKERS = NUM_CORES * NUM_SUBCORES

G = 256
BLOCKS_PER_H = BATCH // G
NUM_CHUNKS = HIST * BLOCKS_PER_H
PER_WORKER = NUM_CHUNKS // NUM_WORKERS

TPAD = 133


def _body(table_hbm, idxt_hbm, out_hbm, idx_v, rows_v, trans_v, gsem, ssem):
    wid = lax.axis_index("s") * NUM_CORES + lax.axis_index("c")
    c0_iotas = [
        jnp.arange(c0, c0 + 16, dtype=jnp.int32) for c0 in range(0, EMBED_DIM, 16)
    ]

    def chunk_of(g):
        c = wid * PER_WORKER + g
        return c // BLOCKS_PER_H, c % BLOCKS_PER_H

    def gather_start(g, buf):
        h, blk = chunk_of(g)
        pltpu.sync_copy(idxt_hbm.at[h, pl.ds(blk * G, G)], idx_v.at[buf])
        pltpu.async_copy(table_hbm.at[idx_v.at[buf]], rows_v.at[buf], gsem)

    def gather_wait():
        pltpu.make_async_copy(table_hbm.at[pl.ds(0, G)], rows_v.at[0], gsem).wait()

    def transpose_chunk(buf):
        def tbody(r, _):
            bbb = r // 128
            bs = r % 128
            col_v = jnp.full((16,), bs, jnp.int32)
            row_base = bbb * 64
            for k in range(EMBED_DIM // 16):
                vals = rows_v[buf, r, pl.ds(k * 16, 16)]
                plsc.store_scatter(trans_v, [row_base + c0_iotas[k], col_v], vals)
            return 0

        lax.fori_loop(0, G, tbody, 0)

    def store_start(g):
        h, blk = chunk_of(g)
        for bbb in range(G // 128):
            for eb in range(8):
                pltpu.async_copy(
                    trans_v.at[pl.ds(bbb * 64 + eb * 8, 8), pl.ds(0, 128)],
                    out_hbm.at[h, eb, blk * (G // 128) + bbb, :, :],
                    ssem,
                )

    def store_drain():
        def dbody(i, _):
            pltpu.make_async_copy(
                trans_v.at[pl.ds(0, 8), pl.ds(0, 128)],
                out_hbm.at[0, 0, 0, :, :],
                ssem,
            ).wait()
            return 0

        lax.fori_loop(0, 16, dbody, 0)

    gather_start(0, 0)

    def body(g, _):
        buf = lax.rem(g, 2)
        gather_wait()

        @pl.when(g + 1 < PER_WORKER)
        def _():
            gather_start(g + 1, 1 - buf)

        @pl.when(g >= 1)
        def _():
            store_drain()

        transpose_chunk(buf)
        store_start(g)
        return 0

    lax.fori_loop(0, PER_WORKER, body, 0)
    store_drain()


@jax.jit
def _sc_gather(table, idxt):
    mesh = plsc.VectorSubcoreMesh(core_axis_name="c", subcore_axis_name="s")
    kfn = functools.partial(
        pl.kernel,
        mesh=mesh,
        out_type=jax.ShapeDtypeStruct((HIST, 8, BATCH // 128, 8, 128), jnp.float32),
        scratch_types=[
            pltpu.VMEM((2, G), jnp.int32),
            pltpu.VMEM((2, G, EMBED_DIM), jnp.float32),
            pltpu.VMEM(((G // 128) * EMBED_DIM, TPAD), jnp.float32),
            pltpu.SemaphoreType.DMA,
            pltpu.SemaphoreType.DMA,
        ],
        compiler_params=pltpu.CompilerParams(
            use_tc_tiling_on_sc=False, needs_layout_passes=False
        ),
    )(_body)
    return kfn(table, idxt)


TC_BLK = 2048
TC_GRID = -(-VOCAB // TC_BLK)
VPAD = TC_GRID * TC_BLK
TC_H = TC_BLK // 2


def _detile_body(t_ref, o_ref):
    eye = jnp.eye(EMBED_DIM, dtype=jnp.float32)
    y = lax.dot_general(
        t_ref[...], eye, (((0,), (0,)), ((), ())),
        preferred_element_type=jnp.float32,
    )
    o_ref[...] = jnp.concatenate([y[0:TC_H], y[TC_H:TC_BLK]], axis=1)


@jax.jit
def _tc_detile(tt):
    return pl.pallas_call(
        _detile_body,
        grid=(TC_GRID,),
        in_specs=[pl.BlockSpec((EMBED_DIM, TC_BLK), lambda i: (0, i))],
        out_specs=pl.BlockSpec((TC_H, 128), lambda i: (i, 0)),
        out_shape=jax.ShapeDtypeStruct((VPAD // 2, 128), jnp.float32),
    )(tt)


def kernel(index_tensor, embedding_table):
    idxt = index_tensor.T.astype(jnp.int32)
    tlin = _tc_detile(embedding_table.T).reshape(VPAD, EMBED_DIM)
    t = idxt % TC_BLK
    kidx = idxt - t + jnp.where(t < TC_H, 2 * t, 2 * t - (TC_BLK - 1))
    out5 = _sc_gather(tlin, kidx)
    return out5.transpose(2, 4, 0, 1, 3).reshape(BATCH, HIST, EMBED_DIM)


# speedup vs baseline: 1.3886x; 1.3886x over previous
import functools

import jax
import jax.numpy as jnp
from jax import lax
from jax.experimental import pallas as pl
from jax.experimental.pallas import tpu as pltpu
from jax.experimental.pallas import tpu_sc as plsc

BATCH = 16384
HIST = 50
EMBED_DIM = 64
VOCAB = 1000000

NUM_CORES = 2
NUM_SUBCORES = 16
NUM_WORKERS = NUM_CORES * NUM_SUBCORES

G = 256
BLOCKS_PER_H = BATCH // G
NUM_CHUNKS = HIST * BLOCKS_PER_H
PER_WORKER = NUM_CHUNKS // NUM_WORKERS

TPAD = 129


def _body(table_hbm, idxt_hbm, out_hbm, idx_v, rows_v, trans_v, gsem, ssem):
    wid = lax.axis_index("s") * NUM_CORES + lax.axis_index("c")
    c0_iotas = [
        jnp.arange(c0, c0 + 16, dtype=jnp.int32) for c0 in range(0, EMBED_DIM, 16)
    ]

    def chunk_of(g):
        c = wid * PER_WORKER + g
        return c // BLOCKS_PER_H, c % BLOCKS_PER_H

    def gather_start(g, buf):
        h, blk = chunk_of(g)
        pltpu.sync_copy(idxt_hbm.at[h, pl.ds(blk * G, G)], idx_v.at[buf])
        pltpu.async_copy(table_hbm.at[idx_v.at[buf]], rows_v.at[buf], gsem)

    def gather_wait():
        pltpu.make_async_copy(table_hbm.at[pl.ds(0, G)], rows_v.at[0], gsem).wait()

    def transpose_chunk(buf):
        def tbody(rr, _):
            r0 = rr * 2
            bbb = r0 // 128
            row_base = bbb * 64
            for d in range(2):
                r = r0 + d
                col_v = jnp.full((16,), r % 128, jnp.int32)
                for k in range(EMBED_DIM // 16):
                    vals = rows_v[buf, r, pl.ds(k * 16, 16)]
                    plsc.store_scatter(
                        trans_v, [row_base + c0_iotas[k], col_v], vals
                    )
            return 0

        lax.fori_loop(0, G // 2, tbody, 0)

    def store_start(g):
        h, blk = chunk_of(g)
        for bbb in range(G // 128):
            for eb in range(8):
                pltpu.async_copy(
                    trans_v.at[pl.ds(bbb * 64 + eb * 8, 8), pl.ds(0, 128)],
                    out_hbm.at[h, eb, blk * (G // 128) + bbb, :, :],
                    ssem,
                )

    def store_drain():
        def dbody(i, _):
            pltpu.make_async_copy(
                trans_v.at[pl.ds(0, 8), pl.ds(0, 128)],
                out_hbm.at[0, 0, 0, :, :],
                ssem,
            ).wait()
            return 0

        lax.fori_loop(0, 16, dbody, 0)

    gather_start(0, 0)

    def body(g, _):
        buf = lax.rem(g, 2)
        gather_wait()

        @pl.when(g + 1 < PER_WORKER)
        def _():
            gather_start(g + 1, 1 - buf)

        @pl.when(g >= 1)
        def _():
            store_drain()

        transpose_chunk(buf)
        store_start(g)
        return 0

    lax.fori_loop(0, PER_WORKER, body, 0)
    store_drain()


@jax.jit
def _sc_gather(table, idxt):
    mesh = plsc.VectorSubcoreMesh(core_axis_name="c", subcore_axis_name="s")
    kfn = functools.partial(
        pl.kernel,
        mesh=mesh,
        out_type=jax.ShapeDtypeStruct((HIST, 8, BATCH // 128, 8, 128), jnp.float32),
        scratch_types=[
            pltpu.VMEM((2, G), jnp.int32),
            pltpu.VMEM((2, G, EMBED_DIM), jnp.float32),
            pltpu.VMEM(((G // 128) * EMBED_DIM, TPAD), jnp.float32),
            pltpu.SemaphoreType.DMA,
            pltpu.SemaphoreType.DMA,
        ],
        compiler_params=pltpu.CompilerParams(
            use_tc_tiling_on_sc=False, needs_layout_passes=False
        ),
    )(_body)
    return kfn(table, idxt)


TC_BLK = 8192
TC_GRID = -(-VOCAB // TC_BLK)
VPAD = TC_GRID * TC_BLK
TC_H = TC_BLK // 2


def _detile_body(t_ref, o_ref):
    y = t_ref[...].T
    o_ref[...] = jnp.concatenate([y[0:TC_H], y[TC_H:TC_BLK]], axis=1)


@jax.jit
def _tc_detile(tt):
    return pl.pallas_call(
        _detile_body,
        grid=(TC_GRID,),
        in_specs=[pl.BlockSpec((EMBED_DIM, TC_BLK), lambda i: (0, i))],
        out_specs=pl.BlockSpec((TC_H, 128), lambda i: (i, 0)),
        out_shape=jax.ShapeDtypeStruct((VPAD // 2, 128), jnp.float32),
    )(tt)


def kernel(index_tensor, embedding_table):
    idxt = index_tensor.T.astype(jnp.int32)
    tlin = _tc_detile(embedding_table.T).reshape(VPAD, EMBED_DIM)
    t = idxt % TC_BLK
    kidx = idxt - t + jnp.where(t < TC_H, 2 * t, 2 * t - (TC_BLK - 1))
    out5 = _sc_gather(tlin, kidx)
    return out5.transpose(2, 4, 0, 1, 3).reshape(BATCH, HIST, EMBED_DIM)
